# Initial kernel scaffold; baseline (speedup 1.0000x reference)
import jax
import jax.numpy as jnp
from jax.experimental import pallas as pl


def kernel(features, neighbor_idx, W1, gamma1, beta1, W2, gamma2, beta2):
    raise NotImplementedError("write your pallas kernel here")


# speedup vs baseline: 2.6133x; 2.6133x over previous
import functools

import jax
import jax.numpy as jnp
import numpy as np
from jax import lax
from jax.experimental import pallas as pl
from jax.experimental.pallas import tpu as pltpu
from jax.experimental.pallas import tpu_sc as plsc

_N = 50000
_C = 128
_K = 9
_NC = 2
_NS = 16
_NW = _NC * _NS
_NPW = 1568
_NP = _NW * _NPW
_B = 32
_NBLK = _NPW // _B
_TN = 400
_EPS = 1e-5
_C16 = _C // 16


def _split_store(xw, y_ref):
    for k in range(_K):
        y_ref[k] = xw[:, k * _C:(k + 1) * _C]


def _bmm_body(x_ref, w_ref, y_ref):
    xw = jnp.dot(x_ref[...], w_ref[...], preferred_element_type=jnp.float32)
    _split_store(xw, y_ref)


def _bmm(x, wcat):
    return pl.pallas_call(
        _bmm_body,
        grid=(_N // _TN,),
        in_specs=[pl.BlockSpec((_TN, _C), lambda i: (i, 0)),
                  pl.BlockSpec((_C, _K * _C), lambda i: (0, 0))],
        out_specs=pl.BlockSpec((_K, _TN, _C), lambda i: (0, i, 0)),
        out_shape=jax.ShapeDtypeStruct((_K, _N, _C), jnp.float32),
    )(x, wcat)


def _stats(p_block):
    s = jnp.sum(p_block[0], axis=0, keepdims=True)
    ss = jnp.sum(p_block[1], axis=0, keepdims=True)
    mean = s / _N
    var = ss / _N - mean * mean
    rstd = lax.rsqrt(var + _EPS)
    return mean, rstd


def _bn_bmm_body(h_ref, p_ref, g_ref, b_ref, w_ref, y_ref):
    mean, rstd = _stats(p_ref[...])
    xn = (h_ref[...] - mean) * (rstd * g_ref[...]) + b_ref[...]
    xn = jnp.maximum(xn, 0.0)
    xw = jnp.dot(xn, w_ref[...], preferred_element_type=jnp.float32)
    _split_store(xw, y_ref)


def _bn_bmm(h, p, gamma, beta, wcat):
    return pl.pallas_call(
        _bn_bmm_body,
        grid=(_N // _TN,),
        in_specs=[pl.BlockSpec((_TN, _C), lambda i: (i, 0)),
                  pl.BlockSpec((2, _NW, _C), lambda i: (0, 0, 0)),
                  pl.BlockSpec((1, _C), lambda i: (0, 0)),
                  pl.BlockSpec((1, _C), lambda i: (0, 0)),
                  pl.BlockSpec((_C, _K * _C), lambda i: (0, 0))],
        out_specs=pl.BlockSpec((_K, _TN, _C), lambda i: (0, i, 0)),
        out_shape=jax.ShapeDtypeStruct((_K, _N, _C), jnp.float32),
    )(h, p, gamma, beta, wcat)


def _final_body(h_ref, p_ref, g_ref, b_ref, f_ref, o_ref):
    mean, rstd = _stats(p_ref[...])
    xn = (h_ref[...] - mean) * (rstd * g_ref[...]) + b_ref[...]
    o_ref[...] = jnp.maximum(xn + f_ref[...], 0.0)


def _final(h, p, gamma, beta, feat):
    return pl.pallas_call(
        _final_body,
        grid=(_N // _TN,),
        in_specs=[pl.BlockSpec((_TN, _C), lambda i: (i, 0)),
                  pl.BlockSpec((2, _NW, _C), lambda i: (0, 0, 0)),
                  pl.BlockSpec((1, _C), lambda i: (0, 0)),
                  pl.BlockSpec((1, _C), lambda i: (0, 0)),
                  pl.BlockSpec((_TN, _C), lambda i: (i, 0))],
        out_specs=pl.BlockSpec((_TN, _C), lambda i: (i, 0)),
        out_shape=jax.ShapeDtypeStruct((_N, _C), jnp.float32),
    )(h, p, gamma, beta, feat)


@functools.cache
def _make_gather_sum():
    return functools.partial(
        pl.kernel,
        mesh=plsc.VectorSubcoreMesh(core_axis_name="c", subcore_axis_name="s"),
        out_type=(jax.ShapeDtypeStruct((_NP, _C), jnp.float32),
                  jax.ShapeDtypeStruct((2, _NW, _C), jnp.float32)),
        scratch_types=[
            pltpu.VMEM((_K * _B, _C), jnp.float32),
            pltpu.VMEM((_K * _B, _C), jnp.float32),
            pltpu.VMEM((_B, _C), jnp.float32),
            pltpu.VMEM((_B, _C), jnp.float32),
            pltpu.VMEM((_K * _B,), jnp.int32),
            pltpu.VMEM((_K * _B,), jnp.int32),
            pltpu.VMEM((2, _C), jnp.float32),
            pltpu.SemaphoreType.DMA,
            pltpu.SemaphoreType.DMA,
            pltpu.SemaphoreType.DMA,
            pltpu.SemaphoreType.DMA,
            pltpu.SemaphoreType.DMA,
            pltpu.SemaphoreType.DMA,
        ],
    )(_gather_sum_body)


def _gather_sum_body(y_hbm, idx_hbm, h_out, p_out,
                     bank0, bank1, stg0, stg1, idx0, idx1, statbuf,
                     sb0, sb1, sw0, sw1, si0, si1):
    wid = lax.axis_index("s") * _NC + lax.axis_index("c")
    base = wid * _NPW
    ibase = wid * (_NBLK * _K * _B)

    banks = (bank0, bank1)
    stgs = (stg0, stg1)
    idxs = (idx0, idx1)
    bsems = (sb0, sb1)
    wsems = (sw0, sw1)
    isems = (si0, si1)

    def issue_gathers(par):
        bank, idxb, sem = banks[par], idxs[par], bsems[par]
        for k in range(_K):
            pltpu.async_copy(y_hbm.at[idxb.at[pl.ds(k * _B, _B)]],
                             bank.at[pl.ds(k * _B, _B)], sem)

    def drain_bank(par):
        pltpu.make_async_copy(y_hbm.at[pl.ds(0, _K * _B)], banks[par],
                              bsems[par]).wait()

    def fetch_idx(par, b):
        pltpu.async_copy(idx_hbm.at[pl.ds(ibase + b * (_K * _B), _K * _B)],
                         idxs[par], isems[par])

    def drain_idx(par):
        pltpu.make_async_copy(idx_hbm.at[pl.ds(0, _K * _B)], idxs[par],
                              isems[par]).wait()

    def drain_wb(par):
        pltpu.make_async_copy(h_out.at[pl.ds(0, _B)], stgs[par],
                              wsems[par]).wait()

    def make_row(bank, stg, with_stats):
        def row(r, st):
            s0, s1 = st
            n0, n1 = [], []
            for j in range(_C16):
                sl = pl.ds(j * 16, 16)
                a = bank[r, sl]
                for k in range(1, _K):
                    a = a + bank[k * _B + r, sl]
                stg[r, sl] = a
                if with_stats:
                    n0.append(s0[j] + a)
                    n1.append(s1[j] + a * a)
            if with_stats:
                return (tuple(n0), tuple(n1))
            return st
        return row

    def process(b, par, stats):
        pos = base + b * _B
        drain_bank(par)
        nxt = 1 - par

        @pl.when(b + 1 < _NBLK)
        def _():
            drain_idx(nxt)
            issue_gathers(nxt)

        @pl.when(b + 2 < _NBLK)
        def _():
            fetch_idx(par, b + 2)

        @pl.when(b >= 2)
        def _():
            drain_wb(par)

        nvalid = jnp.minimum(_B, jnp.maximum(_N - pos, 0))
        stats = lax.fori_loop(0, nvalid,
                              make_row(banks[par], stgs[par], True), stats)
        lax.fori_loop(nvalid, _B,
                      make_row(banks[par], stgs[par], False), stats)
        pltpu.async_copy(stgs[par], h_out.at[pl.ds(pos, _B)], wsems[par])
        return stats

    pltpu.sync_copy(idx_hbm.at[pl.ds(ibase, _K * _B)], idx0)
    issue_gathers(0)
    fetch_idx(1, 1)

    def pair(i, stats):
        stats = process(2 * i, 0, stats)
        return process(2 * i + 1, 1, stats)

    zeros = tuple(jnp.zeros((16,), jnp.float32) for _ in range(_C16))
    stats = lax.fori_loop(0, _NBLK // 2, pair, (zeros, zeros))
    stats = process(_NBLK - 1, 0, stats)

    drain_wb(0)
    drain_wb(1)
    for c in range(_C16):
        sl = pl.ds(c * 16, 16)
        statbuf[0, sl] = stats[0][c]
        statbuf[1, sl] = stats[1][c]
    pltpu.sync_copy(statbuf.at[0], p_out.at[0, wid])
    pltpu.sync_copy(statbuf.at[1], p_out.at[1, wid])


def kernel(features, neighbor_idx, W1, gamma1, beta1, W2, gamma2, beta2):
    idx32 = neighbor_idx.astype(jnp.int32)
    idx_off = idx32 + (jnp.arange(_K, dtype=jnp.int32) * _N)[:, None]
    idx_p = jnp.zeros((_K, _NP), jnp.int32).at[:, :_N].set(idx_off)
    idx_p = (idx_p.reshape(_K, _NW, _NBLK, _B)
             .transpose(1, 2, 0, 3).reshape(_NW * _NBLK * _K * _B))

    w1cat = jnp.transpose(W1, (1, 0, 2)).reshape(_C, _K * _C)
    w2cat = jnp.transpose(W2, (1, 0, 2)).reshape(_C, _K * _C)
    g1 = gamma1.reshape(1, _C)
    b1 = beta1.reshape(1, _C)
    g2 = gamma2.reshape(1, _C)
    b2 = beta2.reshape(1, _C)

    gather_sum = _make_gather_sum()
    y1 = _bmm(features, w1cat).reshape(_K * _N, _C)
    h1, p1 = gather_sum(y1, idx_p)
    y2 = _bn_bmm(h1, p1, g1, b1, w2cat).reshape(_K * _N, _C)
    h2, p2 = gather_sum(y2, idx_p)
    return _final(h2, p2, g2, b2, features)


# speedup vs baseline: 3.1023x; 1.1871x over previous
import functools

import jax
import jax.numpy as jnp
import numpy as np
from jax import lax
from jax.experimental import pallas as pl
from jax.experimental.pallas import tpu as pltpu
from jax.experimental.pallas import tpu_sc as plsc

_N = 50000
_C = 128
_K = 9
_NC = 2
_NS = 16
_NW = _NC * _NS
_NPW = 1568
_NP = _NW * _NPW
_B = 32
_NBT = _NP // _B
_NBLK0 = 58
_NBLK1 = 40
_TN = 1000
_EPS = 1e-5
_C16 = _C // 16


def _split_store(xw, y_ref):
    for k in range(_K):
        y_ref[k] = xw[:, k * _C:(k + 1) * _C]


def _bmm_body(x_ref, w_ref, y_ref):
    xw = jnp.dot(x_ref[...], w_ref[...], preferred_element_type=jnp.float32)
    _split_store(xw, y_ref)


def _bmm(x, wcat):
    return pl.pallas_call(
        _bmm_body,
        grid=(_N // _TN,),
        in_specs=[pl.BlockSpec((_TN, _C), lambda i: (i, 0)),
                  pl.BlockSpec((_C, _K * _C), lambda i: (0, 0))],
        out_specs=pl.BlockSpec((_K, _TN, _C), lambda i: (0, i, 0)),
        out_shape=jax.ShapeDtypeStruct((_K, _N, _C), jnp.float32),
    )(x, wcat)


def _stats(p_block):
    s = jnp.sum(p_block[0], axis=0, keepdims=True)
    ss = jnp.sum(p_block[1], axis=0, keepdims=True)
    mean = s / _N
    var = ss / _N - mean * mean
    rstd = lax.rsqrt(var + _EPS)
    return mean, rstd


def _bn_bmm_body(h_ref, p_ref, g_ref, b_ref, w_ref, y_ref):
    mean, rstd = _stats(p_ref[...])
    xn = (h_ref[...] - mean) * (rstd * g_ref[...]) + b_ref[...]
    xn = jnp.maximum(xn, 0.0)
    xw = jnp.dot(xn, w_ref[...], preferred_element_type=jnp.float32)
    _split_store(xw, y_ref)


def _bn_bmm(h, p, gamma, beta, wcat):
    return pl.pallas_call(
        _bn_bmm_body,
        grid=(_N // _TN,),
        in_specs=[pl.BlockSpec((_TN, _C), lambda i: (i, 0)),
                  pl.BlockSpec((2, _NW, _C), lambda i: (0, 0, 0)),
                  pl.BlockSpec((1, _C), lambda i: (0, 0)),
                  pl.BlockSpec((1, _C), lambda i: (0, 0)),
                  pl.BlockSpec((_C, _K * _C), lambda i: (0, 0))],
        out_specs=pl.BlockSpec((_K, _TN, _C), lambda i: (0, i, 0)),
        out_shape=jax.ShapeDtypeStruct((_K, _N, _C), jnp.float32),
    )(h, p, gamma, beta, wcat)


def _final_body(h_ref, p_ref, g_ref, b_ref, f_ref, o_ref):
    mean, rstd = _stats(p_ref[...])
    xn = (h_ref[...] - mean) * (rstd * g_ref[...]) + b_ref[...]
    o_ref[...] = jnp.maximum(xn + f_ref[...], 0.0)


def _final(h, p, gamma, beta, feat):
    return pl.pallas_call(
        _final_body,
        grid=(_N // _TN,),
        in_specs=[pl.BlockSpec((_TN, _C), lambda i: (i, 0)),
                  pl.BlockSpec((2, _NW, _C), lambda i: (0, 0, 0)),
                  pl.BlockSpec((1, _C), lambda i: (0, 0)),
                  pl.BlockSpec((1, _C), lambda i: (0, 0)),
                  pl.BlockSpec((_TN, _C), lambda i: (i, 0))],
        out_specs=pl.BlockSpec((_TN, _C), lambda i: (i, 0)),
        out_shape=jax.ShapeDtypeStruct((_N, _C), jnp.float32),
    )(h, p, gamma, beta, feat)


@functools.cache
def _make_gather_sum():
    return functools.partial(
        pl.kernel,
        mesh=plsc.VectorSubcoreMesh(core_axis_name="c", subcore_axis_name="s"),
        out_type=(jax.ShapeDtypeStruct((_NP, _C), jnp.float32),
                  jax.ShapeDtypeStruct((2, _NW, _C), jnp.float32)),
        scratch_types=[
            pltpu.VMEM((_K * _B, _C), jnp.float32),
            pltpu.VMEM((_K * _B, _C), jnp.float32),
            pltpu.VMEM((_B, _C), jnp.float32),
            pltpu.VMEM((_B, _C), jnp.float32),
            pltpu.VMEM((_K * _B,), jnp.int32),
            pltpu.VMEM((_K * _B,), jnp.int32),
            pltpu.VMEM((2, _C), jnp.float32),
            pltpu.SemaphoreType.DMA,
            pltpu.SemaphoreType.DMA,
            pltpu.SemaphoreType.DMA,
            pltpu.SemaphoreType.DMA,
            pltpu.SemaphoreType.DMA,
            pltpu.SemaphoreType.DMA,
        ],
    )(_gather_sum_body)


def _gather_sum_body(y_hbm, idx_hbm, h_out, p_out,
                     bank0, bank1, stg0, stg1, idx0, idx1, statbuf,
                     sb0, sb1, sw0, sw1, si0, si1):
    cid = lax.axis_index("c")
    sid = lax.axis_index("s")
    wid = sid * _NC + cid
    nblk = jnp.where(cid == 0, _NBLK0, _NBLK1)
    blk0 = jnp.where(cid == 0, sid * _NBLK0,
                     _NS * _NBLK0 + sid * _NBLK1)
    base = blk0 * _B
    ibase = blk0 * (_K * _B)

    banks = (bank0, bank1)
    stgs = (stg0, stg1)
    idxs = (idx0, idx1)
    bsems = (sb0, sb1)
    wsems = (sw0, sw1)
    isems = (si0, si1)

    def issue_gathers(par):
        bank, idxb, sem = banks[par], idxs[par], bsems[par]
        for k in range(_K):
            pltpu.async_copy(y_hbm.at[idxb.at[pl.ds(k * _B, _B)]],
                             bank.at[pl.ds(k * _B, _B)], sem)

    def drain_bank(par):
        pltpu.make_async_copy(y_hbm.at[pl.ds(0, _K * _B)], banks[par],
                              bsems[par]).wait()

    def fetch_idx(par, b):
        pltpu.async_copy(idx_hbm.at[pl.ds(ibase + b * (_K * _B), _K * _B)],
                         idxs[par], isems[par])

    def drain_idx(par):
        pltpu.make_async_copy(idx_hbm.at[pl.ds(0, _K * _B)], idxs[par],
                              isems[par]).wait()

    def drain_wb(par):
        pltpu.make_async_copy(h_out.at[pl.ds(0, _B)], stgs[par],
                              wsems[par]).wait()

    def make_row(bank, stg, with_stats):
        def row(r, st):
            s0, s1 = st
            n0, n1 = [], []
            for j in range(_C16):
                sl = pl.ds(j * 16, 16)
                a = bank[r, sl]
                for k in range(1, _K):
                    a = a + bank[k * _B + r, sl]
                stg[r, sl] = a
                if with_stats:
                    n0.append(s0[j] + a)
                    n1.append(s1[j] + a * a)
            if with_stats:
                return (tuple(n0), tuple(n1))
            return st
        return row

    def process(b, par, stats):
        pos = base + b * _B
        drain_bank(par)
        nxt = 1 - par

        @pl.when(b + 1 < nblk)
        def _():
            drain_idx(nxt)
            issue_gathers(nxt)

        @pl.when(b + 2 < nblk)
        def _():
            fetch_idx(par, b + 2)

        @pl.when(b >= 2)
        def _():
            drain_wb(par)

        nvalid = jnp.minimum(_B, jnp.maximum(_N - pos, 0))
        stats = lax.fori_loop(0, nvalid,
                              make_row(banks[par], stgs[par], True), stats)
        lax.fori_loop(nvalid, _B,
                      make_row(banks[par], stgs[par], False), stats)
        pltpu.async_copy(stgs[par], h_out.at[pl.ds(pos, _B)], wsems[par])
        return stats

    pltpu.sync_copy(idx_hbm.at[pl.ds(ibase, _K * _B)], idx0)
    issue_gathers(0)
    fetch_idx(1, 1)

    def pair(i, stats):
        stats = process(2 * i, 0, stats)
        return process(2 * i + 1, 1, stats)

    zeros = tuple(jnp.zeros((16,), jnp.float32) for _ in range(_C16))
    stats = lax.fori_loop(0, nblk // 2, pair, (zeros, zeros))

    drain_wb(0)
    drain_wb(1)
    for c in range(_C16):
        sl = pl.ds(c * 16, 16)
        statbuf[0, sl] = stats[0][c]
        statbuf[1, sl] = stats[1][c]
    pltpu.sync_copy(statbuf.at[0], p_out.at[0, wid])
    pltpu.sync_copy(statbuf.at[1], p_out.at[1, wid])


def kernel(features, neighbor_idx, W1, gamma1, beta1, W2, gamma2, beta2):
    idx32 = neighbor_idx.astype(jnp.int32)
    idx_off = idx32 + (jnp.arange(_K, dtype=jnp.int32) * _N)[:, None]
    idx_p = jnp.zeros((_K, _NP), jnp.int32).at[:, :_N].set(idx_off)
    idx_p = (idx_p.reshape(_K, _NBT, _B)
             .transpose(1, 0, 2).reshape(_NBT * _K * _B))

    w1cat = jnp.transpose(W1, (1, 0, 2)).reshape(_C, _K * _C)
    w2cat = jnp.transpose(W2, (1, 0, 2)).reshape(_C, _K * _C)
    g1 = gamma1.reshape(1, _C)
    b1 = beta1.reshape(1, _C)
    g2 = gamma2.reshape(1, _C)
    b2 = beta2.reshape(1, _C)

    gather_sum = _make_gather_sum()
    y1 = _bmm(features, w1cat).reshape(_K * _N, _C)
    h1, p1 = gather_sum(y1, idx_p)
    y2 = _bn_bmm(h1, p1, g1, b1, w2cat).reshape(_K * _N, _C)
    h2, p2 = gather_sum(y2, idx_p)
    return _final(h2, p2, g2, b2, features)


# speedup vs baseline: 3.1044x; 1.0007x over previous
import functools

import jax
import jax.numpy as jnp
from jax import lax
from jax.experimental import pallas as pl
from jax.experimental.pallas import tpu as pltpu
from jax.experimental.pallas import tpu_sc as plsc

_N = 50000
_C = 128
_K = 9
_NC = 2
_NS = 16
_NW = _NC * _NS
_NPW = 1568
_NP = _NW * _NPW
_B = 32
_NBT = _NP // _B
_NBLK0 = 58
_NBLK1 = 40
_TN = 1000
_EPS = 1e-5
_C16 = _C // 16


def _split_store(xw, y_ref):
    for k in range(_K):
        y_ref[k] = xw[:, k * _C:(k + 1) * _C]


def _bmm_body(x_ref, w_ref, y_ref):
    xw = jnp.dot(x_ref[...], w_ref[...], preferred_element_type=jnp.float32)
    _split_store(xw, y_ref)


def _bmm(x, wcat):
    return pl.pallas_call(
        _bmm_body,
        grid=(_N // _TN,),
        in_specs=[pl.BlockSpec((_TN, _C), lambda i: (i, 0)),
                  pl.BlockSpec((_C, _K * _C), lambda i: (0, 0))],
        out_specs=pl.BlockSpec((_K, _TN, _C), lambda i: (0, i, 0)),
        out_shape=jax.ShapeDtypeStruct((_K, _N, _C), jnp.float32),
    )(x, wcat)


def _stats(p_block):
    s = jnp.sum(p_block[0], axis=0, keepdims=True)
    ss = jnp.sum(p_block[1], axis=0, keepdims=True)
    mean = s / _N
    var = ss / _N - mean * mean
    rstd = lax.rsqrt(var + _EPS)
    return mean, rstd


def _bn_bmm_body(h_ref, p_ref, g_ref, b_ref, w_ref, y_ref):
    mean, rstd = _stats(p_ref[...])
    xn = (h_ref[...] - mean) * (rstd * g_ref[...]) + b_ref[...]
    xn = jnp.maximum(xn, 0.0)
    xw = jnp.dot(xn, w_ref[...], preferred_element_type=jnp.float32)
    _split_store(xw, y_ref)


def _bn_bmm(h, p, gamma, beta, wcat):
    return pl.pallas_call(
        _bn_bmm_body,
        grid=(_N // _TN,),
        in_specs=[pl.BlockSpec((_TN, _C), lambda i: (i, 0)),
                  pl.BlockSpec((2, _NW, _C), lambda i: (0, 0, 0)),
                  pl.BlockSpec((1, _C), lambda i: (0, 0)),
                  pl.BlockSpec((1, _C), lambda i: (0, 0)),
                  pl.BlockSpec((_C, _K * _C), lambda i: (0, 0))],
        out_specs=pl.BlockSpec((_K, _TN, _C), lambda i: (0, i, 0)),
        out_shape=jax.ShapeDtypeStruct((_K, _N, _C), jnp.float32),
    )(h, p, gamma, beta, wcat)


def _final_body(h_ref, p_ref, g_ref, b_ref, f_ref, o_ref):
    mean, rstd = _stats(p_ref[...])
    xn = (h_ref[...] - mean) * (rstd * g_ref[...]) + b_ref[...]
    o_ref[...] = jnp.maximum(xn + f_ref[...], 0.0)


def _final(h, p, gamma, beta, feat):
    return pl.pallas_call(
        _final_body,
        grid=(_N // _TN,),
        in_specs=[pl.BlockSpec((_TN, _C), lambda i: (i, 0)),
                  pl.BlockSpec((2, _NW, _C), lambda i: (0, 0, 0)),
                  pl.BlockSpec((1, _C), lambda i: (0, 0)),
                  pl.BlockSpec((1, _C), lambda i: (0, 0)),
                  pl.BlockSpec((_TN, _C), lambda i: (i, 0))],
        out_specs=pl.BlockSpec((_TN, _C), lambda i: (i, 0)),
        out_shape=jax.ShapeDtypeStruct((_N, _C), jnp.float32),
    )(h, p, gamma, beta, feat)


@functools.cache
def _make_gather_sum():
    return functools.partial(
        pl.kernel,
        mesh=plsc.VectorSubcoreMesh(core_axis_name="c", subcore_axis_name="s"),
        out_type=(jax.ShapeDtypeStruct((_NP, _C), jnp.float32),
                  jax.ShapeDtypeStruct((2, _NW, _C), jnp.float32)),
        scratch_types=[
            pltpu.VMEM((_K * _B, _C), jnp.float32),
            pltpu.VMEM((_K * _B, _C), jnp.float32),
            pltpu.VMEM((_B, _C), jnp.float32),
            pltpu.VMEM((_B, _C), jnp.float32),
            pltpu.VMEM((_K * _B,), jnp.int32),
            pltpu.VMEM((_K * _B,), jnp.int32),
            pltpu.VMEM((2, _C), jnp.float32),
            pltpu.SemaphoreType.DMA,
            pltpu.SemaphoreType.DMA,
            pltpu.SemaphoreType.DMA,
            pltpu.SemaphoreType.DMA,
            pltpu.SemaphoreType.DMA,
            pltpu.SemaphoreType.DMA,
        ],
    )(_gather_sum_body)


def _gather_sum_body(y_hbm, idx_hbm, h_out, p_out,
                     bank0, bank1, stg0, stg1, idx0, idx1, statbuf,
                     sb0, sb1, sw0, sw1, si0, si1):
    cid = lax.axis_index("c")
    sid = lax.axis_index("s")
    wid = sid * _NC + cid
    nblk = jnp.where(cid == 0, _NBLK0, _NBLK1)
    blk0 = jnp.where(cid == 0, sid * _NBLK0,
                     _NS * _NBLK0 + sid * _NBLK1)
    base = blk0 * _B
    ibase = blk0 * (_K * _B)

    banks = (bank0, bank1)
    stgs = (stg0, stg1)
    idxs = (idx0, idx1)
    bsems = (sb0, sb1)
    wsems = (sw0, sw1)
    isems = (si0, si1)

    def issue_gathers(par):
        bank, idxb, sem = banks[par], idxs[par], bsems[par]
        for k in range(_K):
            pltpu.async_copy(y_hbm.at[idxb.at[pl.ds(k * _B, _B)]],
                             bank.at[pl.ds(k * _B, _B)], sem)

    def drain_bank(par):
        pltpu.make_async_copy(y_hbm.at[pl.ds(0, _K * _B)], banks[par],
                              bsems[par]).wait()

    def fetch_idx(par, b):
        pltpu.async_copy(idx_hbm.at[pl.ds(ibase + b * (_K * _B), _K * _B)],
                         idxs[par], isems[par])

    def drain_idx(par):
        pltpu.make_async_copy(idx_hbm.at[pl.ds(0, _K * _B)], idxs[par],
                              isems[par]).wait()

    def drain_wb(par):
        pltpu.make_async_copy(h_out.at[pl.ds(0, _B)], stgs[par],
                              wsems[par]).wait()

    def make_row(bank, stg, with_stats):
        def row(r, st):
            s0, s1 = st
            n0, n1 = [], []
            for j in range(_C16):
                sl = pl.ds(j * 16, 16)
                a = bank[r, sl]
                for k in range(1, _K):
                    a = a + bank[k * _B + r, sl]
                stg[r, sl] = a
                if with_stats:
                    n0.append(s0[j] + a)
                    n1.append(s1[j] + a * a)
            if with_stats:
                return (tuple(n0), tuple(n1))
            return st
        return row

    def process(b, par, stats):
        pos = base + b * _B
        drain_bank(par)
        nxt = 1 - par

        @pl.when(b + 1 < nblk)
        def _():
            drain_idx(nxt)
            issue_gathers(nxt)

        @pl.when(b + 2 < nblk)
        def _():
            fetch_idx(par, b + 2)

        @pl.when(b >= 2)
        def _():
            drain_wb(par)

        nvalid = jnp.minimum(_B, jnp.maximum(_N - pos, 0))
        stats = lax.fori_loop(0, nvalid,
                              make_row(banks[par], stgs[par], True), stats)
        lax.fori_loop(nvalid, _B,
                      make_row(banks[par], stgs[par], False), stats)
        pltpu.async_copy(stgs[par], h_out.at[pl.ds(pos, _B)], wsems[par])
        return stats

    pltpu.sync_copy(idx_hbm.at[pl.ds(ibase, _K * _B)], idx0)
    issue_gathers(0)
    fetch_idx(1, 1)

    def pair(i, stats):
        stats = process(2 * i, 0, stats)
        return process(2 * i + 1, 1, stats)

    zeros = tuple(jnp.zeros((16,), jnp.float32) for _ in range(_C16))
    stats = lax.fori_loop(0, nblk // 2, pair, (zeros, zeros))

    drain_wb(0)
    drain_wb(1)
    for c in range(_C16):
        sl = pl.ds(c * 16, 16)
        statbuf[0, sl] = stats[0][c]
        statbuf[1, sl] = stats[1][c]
    pltpu.sync_copy(statbuf.at[0], p_out.at[0, wid])
    pltpu.sync_copy(statbuf.at[1], p_out.at[1, wid])


def kernel(features, neighbor_idx, W1, gamma1, beta1, W2, gamma2, beta2):
    idx32 = neighbor_idx.astype(jnp.int32)
    idx_off = idx32 + (jnp.arange(_K, dtype=jnp.int32) * _N)[:, None]
    idx_p = jnp.zeros((_K, _NP), jnp.int32).at[:, :_N].set(idx_off)
    idx_p = (idx_p.reshape(_K, _NBT, _B)
             .transpose(1, 0, 2).reshape(_NBT * _K * _B))

    w1cat = jnp.transpose(W1, (1, 0, 2)).reshape(_C, _K * _C)
    w2cat = jnp.transpose(W2, (1, 0, 2)).reshape(_C, _K * _C)
    g1 = gamma1.reshape(1, _C)
    b1 = beta1.reshape(1, _C)
    g2 = gamma2.reshape(1, _C)
    b2 = beta2.reshape(1, _C)

    gather_sum = _make_gather_sum()
    y1 = _bmm(features, w1cat).reshape(_K * _N, _C)
    h1, p1 = gather_sum(y1, idx_p)
    y2 = _bn_bmm(h1, p1, g1, b1, w2cat).reshape(_K * _N, _C)
    h2, p2 = gather_sum(y2, idx_p)
    return _final(h2, p2, g2, b2, features)


# speedup vs baseline: 3.2768x; 1.0555x over previous
import functools

import jax
import jax.numpy as jnp
from jax import lax
from jax.experimental import pallas as pl
from jax.experimental.pallas import tpu as pltpu
from jax.experimental.pallas import tpu_sc as plsc

_N = 50000
_C = 128
_K = 9
_NC = 2
_NS = 16
_NW = _NC * _NS
_NPW = 1568
_NP = _NW * _NPW
_B = 32
_NBT = _NP // _B
_NBLK0 = 58
_NBLK1 = 40
_TN = 2000
_EPS = 1e-5
_C16 = _C // 16


def _split_store(xw, y_ref):
    for k in range(_K):
        y_ref[k] = xw[:, k * _C:(k + 1) * _C]


def _bmm_body(x_ref, w_ref, y_ref):
    xw = jnp.dot(x_ref[...], w_ref[...], preferred_element_type=jnp.float32)
    _split_store(xw, y_ref)


def _bmm(x, wcat):
    return pl.pallas_call(
        _bmm_body,
        grid=(_N // _TN,),
        in_specs=[pl.BlockSpec((_TN, _C), lambda i: (i, 0)),
                  pl.BlockSpec((_C, _K * _C), lambda i: (0, 0))],
        out_specs=pl.BlockSpec((_K, _TN, _C), lambda i: (0, i, 0)),
        out_shape=jax.ShapeDtypeStruct((_K, _N, _C), jnp.float32),
    )(x, wcat)


def _stats(p_block):
    s = jnp.sum(p_block[0], axis=0, keepdims=True)
    ss = jnp.sum(p_block[1], axis=0, keepdims=True)
    mean = s / _N
    var = ss / _N - mean * mean
    rstd = lax.rsqrt(var + _EPS)
    return mean, rstd


def _bn_bmm_body(h_ref, p_ref, g_ref, b_ref, w_ref, y_ref):
    mean, rstd = _stats(p_ref[...])
    xn = (h_ref[...] - mean) * (rstd * g_ref[...]) + b_ref[...]
    xn = jnp.maximum(xn, 0.0)
    xw = jnp.dot(xn, w_ref[...], preferred_element_type=jnp.float32)
    _split_store(xw, y_ref)


def _bn_bmm(h, p, gamma, beta, wcat):
    return pl.pallas_call(
        _bn_bmm_body,
        grid=(_N // _TN,),
        in_specs=[pl.BlockSpec((_TN, _C), lambda i: (i, 0)),
                  pl.BlockSpec((2, _NW, _C), lambda i: (0, 0, 0)),
                  pl.BlockSpec((1, _C), lambda i: (0, 0)),
                  pl.BlockSpec((1, _C), lambda i: (0, 0)),
                  pl.BlockSpec((_C, _K * _C), lambda i: (0, 0))],
        out_specs=pl.BlockSpec((_K, _TN, _C), lambda i: (0, i, 0)),
        out_shape=jax.ShapeDtypeStruct((_K, _N, _C), jnp.float32),
    )(h, p, gamma, beta, wcat)


def _final_body(h_ref, p_ref, g_ref, b_ref, f_ref, o_ref):
    mean, rstd = _stats(p_ref[...])
    xn = (h_ref[...] - mean) * (rstd * g_ref[...]) + b_ref[...]
    o_ref[...] = jnp.maximum(xn + f_ref[...], 0.0)


def _final(h, p, gamma, beta, feat):
    return pl.pallas_call(
        _final_body,
        grid=(_N // _TN,),
        in_specs=[pl.BlockSpec((_TN, _C), lambda i: (i, 0)),
                  pl.BlockSpec((2, _NW, _C), lambda i: (0, 0, 0)),
                  pl.BlockSpec((1, _C), lambda i: (0, 0)),
                  pl.BlockSpec((1, _C), lambda i: (0, 0)),
                  pl.BlockSpec((_TN, _C), lambda i: (i, 0))],
        out_specs=pl.BlockSpec((_TN, _C), lambda i: (i, 0)),
        out_shape=jax.ShapeDtypeStruct((_N, _C), jnp.float32),
    )(h, p, gamma, beta, feat)


@functools.cache
def _make_gather_sum():
    return functools.partial(
        pl.kernel,
        mesh=plsc.VectorSubcoreMesh(core_axis_name="c", subcore_axis_name="s"),
        out_type=(jax.ShapeDtypeStruct((_NP, _C), jnp.float32),
                  jax.ShapeDtypeStruct((2, _NW, _C), jnp.float32)),
        scratch_types=[
            pltpu.VMEM((_K * _B, _C), jnp.float32),
            pltpu.VMEM((_K * _B, _C), jnp.float32),
            pltpu.VMEM((_B, _C), jnp.float32),
            pltpu.VMEM((_B, _C), jnp.float32),
            pltpu.VMEM((_K * _B,), jnp.int32),
            pltpu.VMEM((_K * _B,), jnp.int32),
            pltpu.VMEM((2, _C), jnp.float32),
            pltpu.SemaphoreType.DMA,
            pltpu.SemaphoreType.DMA,
            pltpu.SemaphoreType.DMA,
            pltpu.SemaphoreType.DMA,
            pltpu.SemaphoreType.DMA,
            pltpu.SemaphoreType.DMA,
        ],
    )(_gather_sum_body)


def _gather_sum_body(y_hbm, idx_hbm, h_out, p_out,
                     bank0, bank1, stg0, stg1, idx0, idx1, statbuf,
                     sb0, sb1, sw0, sw1, si0, si1):
    cid = lax.axis_index("c")
    sid = lax.axis_index("s")
    wid = sid * _NC + cid
    nblk = jnp.where(cid == 0, _NBLK0, _NBLK1)
    blk0 = jnp.where(cid == 0, sid * _NBLK0,
                     _NS * _NBLK0 + sid * _NBLK1)
    base = blk0 * _B
    ibase = blk0 * (_K * _B)

    banks = (bank0, bank1)
    stgs = (stg0, stg1)
    idxs = (idx0, idx1)
    bsems = (sb0, sb1)
    wsems = (sw0, sw1)
    isems = (si0, si1)

    def issue_gathers(par):
        bank, idxb, sem = banks[par], idxs[par], bsems[par]
        for k in range(_K):
            pltpu.async_copy(y_hbm.at[idxb.at[pl.ds(k * _B, _B)]],
                             bank.at[pl.ds(k * _B, _B)], sem)

    def drain_bank(par):
        pltpu.make_async_copy(y_hbm.at[pl.ds(0, _K * _B)], banks[par],
                              bsems[par]).wait()

    def fetch_idx(par, b):
        pltpu.async_copy(idx_hbm.at[pl.ds(ibase + b * (_K * _B), _K * _B)],
                         idxs[par], isems[par])

    def drain_idx(par):
        pltpu.make_async_copy(idx_hbm.at[pl.ds(0, _K * _B)], idxs[par],
                              isems[par]).wait()

    def drain_wb(par):
        pltpu.make_async_copy(h_out.at[pl.ds(0, _B)], stgs[par],
                              wsems[par]).wait()

    def make_row(bank, stg, with_stats):
        def row(r, st):
            s0, s1 = st
            n0, n1 = [], []
            for j in range(_C16):
                sl = pl.ds(j * 16, 16)
                a = bank[r, sl]
                for k in range(1, _K):
                    a = a + bank[k * _B + r, sl]
                stg[r, sl] = a
                if with_stats:
                    n0.append(s0[j] + a)
                    n1.append(s1[j] + a * a)
            if with_stats:
                return (tuple(n0), tuple(n1))
            return st
        return row

    def process(b, par, stats):
        pos = base + b * _B
        drain_bank(par)
        nxt = 1 - par

        @pl.when(b + 1 < nblk)
        def _():
            drain_idx(nxt)
            issue_gathers(nxt)

        @pl.when(b + 2 < nblk)
        def _():
            fetch_idx(par, b + 2)

        @pl.when(b >= 2)
        def _():
            drain_wb(par)

        nvalid = jnp.minimum(_B, jnp.maximum(_N - pos, 0))
        stats = lax.fori_loop(0, nvalid,
                              make_row(banks[par], stgs[par], True), stats)
        lax.fori_loop(nvalid, _B,
                      make_row(banks[par], stgs[par], False), stats)
        pltpu.async_copy(stgs[par], h_out.at[pl.ds(pos, _B)], wsems[par])
        return stats

    pltpu.sync_copy(idx_hbm.at[pl.ds(ibase, _K * _B)], idx0)
    issue_gathers(0)
    fetch_idx(1, 1)

    def pair(i, stats):
        stats = process(2 * i, 0, stats)
        return process(2 * i + 1, 1, stats)

    zeros = tuple(jnp.zeros((16,), jnp.float32) for _ in range(_C16))
    stats = lax.fori_loop(0, nblk // 2, pair, (zeros, zeros))

    drain_wb(0)
    drain_wb(1)
    for c in range(_C16):
        sl = pl.ds(c * 16, 16)
        statbuf[0, sl] = stats[0][c]
        statbuf[1, sl] = stats[1][c]
    pltpu.sync_copy(statbuf.at[0], p_out.at[0, wid])
    pltpu.sync_copy(statbuf.at[1], p_out.at[1, wid])


def kernel(features, neighbor_idx, W1, gamma1, beta1, W2, gamma2, beta2):
    idx32 = neighbor_idx.astype(jnp.int32)
    idx_off = idx32 + (jnp.arange(_K, dtype=jnp.int32) * _N)[:, None]
    idx_p = jnp.zeros((_K, _NP), jnp.int32).at[:, :_N].set(idx_off)
    idx_p = (idx_p.reshape(_K, _NBT, _B)
             .transpose(1, 0, 2).reshape(_NBT * _K * _B))

    w1cat = jnp.transpose(W1, (1, 0, 2)).reshape(_C, _K * _C)
    w2cat = jnp.transpose(W2, (1, 0, 2)).reshape(_C, _K * _C)
    g1 = gamma1.reshape(1, _C)
    b1 = beta1.reshape(1, _C)
    g2 = gamma2.reshape(1, _C)
    b2 = beta2.reshape(1, _C)

    gather_sum = _make_gather_sum()
    y1 = _bmm(features, w1cat).reshape(_K * _N, _C)
    h1, p1 = gather_sum(y1, idx_p)
    y2 = _bn_bmm(h1, p1, g1, b1, w2cat).reshape(_K * _N, _C)
    h2, p2 = gather_sum(y2, idx_p)
    return _final(h2, p2, g2, b2, features)


# speedup vs baseline: 3.3126x; 1.0109x over previous
import functools

import jax
import jax.numpy as jnp
from jax import lax
from jax.experimental import pallas as pl
from jax.experimental.pallas import tpu as pltpu
from jax.experimental.pallas import tpu_sc as plsc

_N = 50000
_C = 128
_K = 9
_NC = 2
_NS = 16
_NW = _NC * _NS
_NPW = 1568
_NP = _NW * _NPW
_B = 32
_NBT = _NP // _B
_NBLK0 = 58
_NBLK1 = 40
_TN = 5000
_EPS = 1e-5
_C16 = _C // 16


def _split_store(xw, y_ref):
    for k in range(_K):
        y_ref[k] = xw[:, k * _C:(k + 1) * _C]


def _bmm_body(x_ref, w_ref, y_ref):
    xw = jnp.dot(x_ref[...], w_ref[...], preferred_element_type=jnp.float32)
    _split_store(xw, y_ref)


def _bmm(x, wcat):
    return pl.pallas_call(
        _bmm_body,
        grid=(_N // _TN,),
        in_specs=[pl.BlockSpec((_TN, _C), lambda i: (i, 0)),
                  pl.BlockSpec((_C, _K * _C), lambda i: (0, 0))],
        out_specs=pl.BlockSpec((_K, _TN, _C), lambda i: (0, i, 0)),
        out_shape=jax.ShapeDtypeStruct((_K, _N, _C), jnp.float32),
    )(x, wcat)


def _stats(p_block):
    s = jnp.sum(p_block[0], axis=0, keepdims=True)
    ss = jnp.sum(p_block[1], axis=0, keepdims=True)
    mean = s / _N
    var = ss / _N - mean * mean
    rstd = lax.rsqrt(var + _EPS)
    return mean, rstd


def _bn_bmm_body(h_ref, p_ref, g_ref, b_ref, w_ref, y_ref):
    mean, rstd = _stats(p_ref[...])
    xn = (h_ref[...] - mean) * (rstd * g_ref[...]) + b_ref[...]
    xn = jnp.maximum(xn, 0.0)
    xw = jnp.dot(xn, w_ref[...], preferred_element_type=jnp.float32)
    _split_store(xw, y_ref)


def _bn_bmm(h, p, gamma, beta, wcat):
    return pl.pallas_call(
        _bn_bmm_body,
        grid=(_N // _TN,),
        in_specs=[pl.BlockSpec((_TN, _C), lambda i: (i, 0)),
                  pl.BlockSpec((2, _NW, _C), lambda i: (0, 0, 0)),
                  pl.BlockSpec((1, _C), lambda i: (0, 0)),
                  pl.BlockSpec((1, _C), lambda i: (0, 0)),
                  pl.BlockSpec((_C, _K * _C), lambda i: (0, 0))],
        out_specs=pl.BlockSpec((_K, _TN, _C), lambda i: (0, i, 0)),
        out_shape=jax.ShapeDtypeStruct((_K, _N, _C), jnp.float32),
    )(h, p, gamma, beta, wcat)


def _final_body(h_ref, p_ref, g_ref, b_ref, f_ref, o_ref):
    mean, rstd = _stats(p_ref[...])
    xn = (h_ref[...] - mean) * (rstd * g_ref[...]) + b_ref[...]
    o_ref[...] = jnp.maximum(xn + f_ref[...], 0.0)


def _final(h, p, gamma, beta, feat):
    return pl.pallas_call(
        _final_body,
        grid=(_N // _TN,),
        in_specs=[pl.BlockSpec((_TN, _C), lambda i: (i, 0)),
                  pl.BlockSpec((2, _NW, _C), lambda i: (0, 0, 0)),
                  pl.BlockSpec((1, _C), lambda i: (0, 0)),
                  pl.BlockSpec((1, _C), lambda i: (0, 0)),
                  pl.BlockSpec((_TN, _C), lambda i: (i, 0))],
        out_specs=pl.BlockSpec((_TN, _C), lambda i: (i, 0)),
        out_shape=jax.ShapeDtypeStruct((_N, _C), jnp.float32),
    )(h, p, gamma, beta, feat)


@functools.cache
def _make_gather_sum():
    return functools.partial(
        pl.kernel,
        mesh=plsc.VectorSubcoreMesh(core_axis_name="c", subcore_axis_name="s"),
        out_type=(jax.ShapeDtypeStruct((_NP, _C), jnp.float32),
                  jax.ShapeDtypeStruct((2, _NW, _C), jnp.float32)),
        scratch_types=[
            pltpu.VMEM((_K * _B, _C), jnp.float32),
            pltpu.VMEM((_K * _B, _C), jnp.float32),
            pltpu.VMEM((_B, _C), jnp.float32),
            pltpu.VMEM((_B, _C), jnp.float32),
            pltpu.VMEM((_K * _B,), jnp.int32),
            pltpu.VMEM((_K * _B,), jnp.int32),
            pltpu.VMEM((2, _C), jnp.float32),
            pltpu.SemaphoreType.DMA,
            pltpu.SemaphoreType.DMA,
            pltpu.SemaphoreType.DMA,
            pltpu.SemaphoreType.DMA,
            pltpu.SemaphoreType.DMA,
            pltpu.SemaphoreType.DMA,
        ],
    )(_gather_sum_body)


def _gather_sum_body(y_hbm, idx_hbm, h_out, p_out,
                     bank0, bank1, stg0, stg1, idx0, idx1, statbuf,
                     sb0, sb1, sw0, sw1, si0, si1):
    cid = lax.axis_index("c")
    sid = lax.axis_index("s")
    wid = sid * _NC + cid
    nblk = jnp.where(cid == 0, _NBLK0, _NBLK1)
    blk0 = jnp.where(cid == 0, sid * _NBLK0,
                     _NS * _NBLK0 + sid * _NBLK1)
    base = blk0 * _B
    ibase = blk0 * (_K * _B)

    banks = (bank0, bank1)
    stgs = (stg0, stg1)
    idxs = (idx0, idx1)
    bsems = (sb0, sb1)
    wsems = (sw0, sw1)
    isems = (si0, si1)

    def issue_gathers(par):
        bank, idxb, sem = banks[par], idxs[par], bsems[par]
        for k in range(_K):
            pltpu.async_copy(y_hbm.at[idxb.at[pl.ds(k * _B, _B)]],
                             bank.at[pl.ds(k * _B, _B)], sem)

    def drain_bank(par):
        pltpu.make_async_copy(y_hbm.at[pl.ds(0, _K * _B)], banks[par],
                              bsems[par]).wait()

    def fetch_idx(par, b):
        pltpu.async_copy(idx_hbm.at[pl.ds(ibase + b * (_K * _B), _K * _B)],
                         idxs[par], isems[par])

    def drain_idx(par):
        pltpu.make_async_copy(idx_hbm.at[pl.ds(0, _K * _B)], idxs[par],
                              isems[par]).wait()

    def drain_wb(par):
        pltpu.make_async_copy(h_out.at[pl.ds(0, _B)], stgs[par],
                              wsems[par]).wait()

    def make_row(bank, stg, with_stats):
        def row(r, st):
            s0, s1 = st
            n0, n1 = [], []
            for j in range(_C16):
                sl = pl.ds(j * 16, 16)
                a = bank[r, sl]
                for k in range(1, _K):
                    a = a + bank[k * _B + r, sl]
                stg[r, sl] = a
                if with_stats:
                    n0.append(s0[j] + a)
                    n1.append(s1[j] + a * a)
            if with_stats:
                return (tuple(n0), tuple(n1))
            return st
        return row

    def process(b, par, stats):
        pos = base + b * _B
        drain_bank(par)
        nxt = 1 - par

        @pl.when(b + 1 < nblk)
        def _():
            drain_idx(nxt)
            issue_gathers(nxt)

        @pl.when(b + 2 < nblk)
        def _():
            fetch_idx(par, b + 2)

        @pl.when(b >= 2)
        def _():
            drain_wb(par)

        nvalid = jnp.minimum(_B, jnp.maximum(_N - pos, 0))
        stats = lax.fori_loop(0, nvalid,
                              make_row(banks[par], stgs[par], True), stats)
        lax.fori_loop(nvalid, _B,
                      make_row(banks[par], stgs[par], False), stats)
        pltpu.async_copy(stgs[par], h_out.at[pl.ds(pos, _B)], wsems[par])
        return stats

    pltpu.sync_copy(idx_hbm.at[pl.ds(ibase, _K * _B)], idx0)
    issue_gathers(0)
    fetch_idx(1, 1)

    def pair(i, stats):
        stats = process(2 * i, 0, stats)
        return process(2 * i + 1, 1, stats)

    zeros = tuple(jnp.zeros((16,), jnp.float32) for _ in range(_C16))
    stats = lax.fori_loop(0, nblk // 2, pair, (zeros, zeros))

    drain_wb(0)
    drain_wb(1)
    for c in range(_C16):
        sl = pl.ds(c * 16, 16)
        statbuf[0, sl] = stats[0][c]
        statbuf[1, sl] = stats[1][c]
    pltpu.sync_copy(statbuf.at[0], p_out.at[0, wid])
    pltpu.sync_copy(statbuf.at[1], p_out.at[1, wid])


def kernel(features, neighbor_idx, W1, gamma1, beta1, W2, gamma2, beta2):
    idx32 = neighbor_idx.astype(jnp.int32)
    idx_off = idx32 + (jnp.arange(_K, dtype=jnp.int32) * _N)[:, None]
    idx_p = jnp.zeros((_K, _NP), jnp.int32).at[:, :_N].set(idx_off)
    idx_p = (idx_p.reshape(_K, _NBT, _B)
             .transpose(1, 0, 2).reshape(_NBT * _K * _B))

    w1cat = jnp.transpose(W1, (1, 0, 2)).reshape(_C, _K * _C)
    w2cat = jnp.transpose(W2, (1, 0, 2)).reshape(_C, _K * _C)
    g1 = gamma1.reshape(1, _C)
    b1 = beta1.reshape(1, _C)
    g2 = gamma2.reshape(1, _C)
    b2 = beta2.reshape(1, _C)

    gather_sum = _make_gather_sum()
    y1 = _bmm(features, w1cat).reshape(_K * _N, _C)
    h1, p1 = gather_sum(y1, idx_p)
    y2 = _bn_bmm(h1, p1, g1, b1, w2cat).reshape(_K * _N, _C)
    h2, p2 = gather_sum(y2, idx_p)
    return _final(h2, p2, g2, b2, features)
